# baseline (speedup 1.0000x reference)
import jax
import jax.numpy as jnp
from jax import lax
from jax.experimental import pallas as pl
from jax.experimental.pallas import tpu as pltpu
from jax.experimental.pallas import tpu_sc as plsc

_N = 100000
_D = 128
_S = 512
_NC = 2
_NS = 16
_NW = _NC * _NS

_C = 128
_NFULL = _N // _C
_TAIL = _N - _NFULL * _C
_NSC = 512
_NTC = _NFULL - _NSC
_K = _NSC // _NW
_REM = _NSC - _K * _NW
_ROWS = _S // _NS


def _seg_body(x_hbm, b_hbm, out_hbm, xbuf, ibuf, xtail, itail, zbuf, rbuf,
              sidx, acc, semx, semi, semsc, semr):
    c = lax.axis_index("c")
    s = lax.axis_index("s")
    r = c * _NS + s
    first = r * _K + jnp.minimum(r, _REM)

    def start(k, b):
        cid = first + k
        base = cid * _C
        hx = pltpu.async_copy(x_hbm.at[pl.ds(base, _C)], xbuf.at[b], semx.at[b])
        hi = pltpu.async_copy(b_hbm.at[pl.ds(base, _C)], ibuf.at[b], semi.at[b])
        return hx, hi

    hin = [None, None]
    hin[0] = start(0, 0)

    zrow = jnp.zeros((16,), jnp.float32)
    for i in range(_ROWS):
        for f in range(_D // 16):
            zbuf[i, pl.ds(f * 16, 16)] = zrow
    for i in range(_S // _ROWS):
        pltpu.sync_copy(zbuf, acc.at[s, pl.ds(i * _ROWS, _ROWS)])

    hs = [None, None]
    for k in range(_K):
        b = k & 1
        nb = 1 - b
        hx, hi = hin[b]
        hx.wait()
        hi.wait()
        if k + 1 < _K:
            if hs[nb] is not None:
                hs[nb].wait()
            hin[nb] = start(k + 1, nb)
        hs[b] = pltpu.async_copy(xbuf.at[b], acc.at[s].at[ibuf.at[b]],
                                 semsc.at[b], add=True)
    for b in range(2):
        if hs[b] is not None:
            hs[b].wait()

    @pl.when(r < _REM)
    def _():
        base = (first + _K) * _C
        pltpu.sync_copy(x_hbm.at[pl.ds(base, _C)], xbuf.at[0])
        pltpu.sync_copy(b_hbm.at[pl.ds(base, _C)], ibuf.at[0])
        pltpu.sync_copy(xbuf.at[0], acc.at[s].at[ibuf.at[0]], add=True)

    @pl.when(r == _NW - 1)
    def _():
        base = _NFULL * _C
        pltpu.sync_copy(x_hbm.at[pl.ds(base, _TAIL)], xtail)
        pltpu.sync_copy(b_hbm.at[pl.ds(base, _TAIL)], itail)
        pltpu.sync_copy(xtail, acc.at[s].at[itail], add=True)

    plsc.subcore_barrier()

    stripe = pl.ds(s * _ROWS, _ROWS)
    iota = lax.iota(jnp.int32, 16)
    for g in range(_ROWS // 16):
        sidx[pl.ds(g * 16, 16)] = s * _ROWS + g * 16 + iota
    hrp = [None, None]
    hrp[0] = pltpu.async_copy(acc.at[0, stripe], rbuf.at[0], semr.at[0])
    hrp[1] = pltpu.async_copy(acc.at[1, stripe], rbuf.at[1], semr.at[1])
    for t in range(2, _NS):
        b = t & 1
        hrp[b].wait()
        hrp[b] = pltpu.async_copy(acc.at[t].at[sidx], rbuf.at[b],
                                  semr.at[b], add=True)
    hrp[0].wait()
    hrp[1].wait()

    def _row(i, _):
        for f in range(_D // 16):
            zbuf[i, pl.ds(f * 16, 16)] = (
                rbuf[0, i, pl.ds(f * 16, 16)] + rbuf[1, i, pl.ds(f * 16, 16)])
        return _

    lax.fori_loop(0, _ROWS, _row, 0)
    pltpu.sync_copy(zbuf, out_hbm.at[pl.ds(c * _S + s * _ROWS, _ROWS)])


_mesh = plsc.VectorSubcoreMesh(
    core_axis_name="c", subcore_axis_name="s",
    num_cores=_NC, num_subcores=_NS)

_seg_sum = pl.kernel(
    _seg_body,
    out_type=jax.ShapeDtypeStruct((_NC * _S, _D), jnp.float32),
    mesh=_mesh,
    scratch_types=[
        pltpu.VMEM((2, _C, _D), jnp.float32),
        pltpu.VMEM((2, _C), jnp.int32),
        pltpu.VMEM((_TAIL, _D), jnp.float32),
        pltpu.VMEM((_TAIL,), jnp.int32),
        pltpu.VMEM((_ROWS, _D), jnp.float32),
        pltpu.VMEM((2, _ROWS, _D), jnp.float32),
        pltpu.VMEM((_ROWS,), jnp.int32),
        pltpu.VMEM_SHARED((_NS, _S, _D), jnp.float32),
        pltpu.SemaphoreType.DMA((2,)),
        pltpu.SemaphoreType.DMA((2,)),
        pltpu.SemaphoreType.DMA((2,)),
        pltpu.SemaphoreType.DMA((2,)),
    ],
)


def _tc_body(i_ref, x_ref, o_ref):
    i = pl.program_id(0)
    ids = i_ref[i + _NSC, :]
    seg = lax.broadcasted_iota(jnp.int32, (_S, _C), 0)
    oh = (seg == ids[None, :]).astype(jnp.float32)
    part = jnp.dot(oh, x_ref[...], preferred_element_type=jnp.float32,
                   precision=lax.Precision.HIGHEST)

    @pl.when(i == 0)
    def _():
        o_ref[...] = part

    @pl.when(i > 0)
    def _():
        o_ref[...] = o_ref[...] + part


_tc_seg_sum = pl.pallas_call(
    _tc_body,
    grid=(_NTC,),
    out_shape=jax.ShapeDtypeStruct((_S, _D), jnp.float32),
    in_specs=[
        pl.BlockSpec(memory_space=pltpu.VMEM),
        pl.BlockSpec((_C, _D), lambda i: (i + _NSC, 0)),
    ],
    out_specs=pl.BlockSpec((_S, _D), lambda i: (0, 0)),
)


def _combine_body(p_ref, t_ref, b_ref, o_ref):
    o_ref[...] = p_ref[:_S, :] + p_ref[_S:, :] + t_ref[...] + b_ref[0]


def kernel(x, batch, bias):
    b32 = batch.astype(jnp.int32)
    partials = _seg_sum(x, b32)
    tc_part = _tc_seg_sum(b32[:_NFULL * _C].reshape(_NFULL, _C), x)
    bias_v = jnp.asarray(bias, jnp.float32).reshape(1)
    return pl.pallas_call(
        _combine_body,
        out_shape=jax.ShapeDtypeStruct((_S, _D), jnp.float32),
        in_specs=[
            pl.BlockSpec(memory_space=pltpu.VMEM),
            pl.BlockSpec(memory_space=pltpu.VMEM),
            pl.BlockSpec(memory_space=pltpu.SMEM),
        ],
        out_specs=pl.BlockSpec(memory_space=pltpu.VMEM),
    )(partials, tc_part, bias_v)


# speedup vs baseline: 2.9623x; 2.9623x over previous
import jax
import jax.numpy as jnp
from jax import lax
from jax.experimental import pallas as pl
from jax.experimental.pallas import tpu as pltpu
from jax.experimental.pallas import tpu_sc as plsc

_N = 100000
_D = 128
_S = 512
_NC = 2
_NS = 16
_NW = _NC * _NS

_C = 128
_NFULL = _N // _C
_TAIL = _N - _NFULL * _C
_K = _NFULL // _NW
_REM = _NFULL - _K * _NW
_ROWS = _S // _NS


def _seg_body(x_hbm, b_hbm, out_hbm, xbuf, ibuf, xtail, itail, zbuf, rbuf,
              sidx, acc, semx, semi, semsc, semr):
    c = lax.axis_index("c")
    s = lax.axis_index("s")
    r = c * _NS + s
    first = r * _K + jnp.minimum(r, _REM)

    def start(k, b):
        cid = first + k
        base = cid * _C
        hx = pltpu.async_copy(x_hbm.at[pl.ds(base, _C)], xbuf.at[b], semx.at[b])
        hi = pltpu.async_copy(b_hbm.at[pl.ds(base, _C)], ibuf.at[b], semi.at[b])
        return hx, hi

    hin = [None, None]
    hin[0] = start(0, 0)

    zrow = jnp.zeros((16,), jnp.float32)
    for i in range(_ROWS):
        for f in range(_D // 16):
            zbuf[i, pl.ds(f * 16, 16)] = zrow
    for i in range(_S // _ROWS):
        pltpu.sync_copy(zbuf, acc.at[s, pl.ds(i * _ROWS, _ROWS)])

    hs = [None, None]
    for k in range(_K):
        b = k & 1
        nb = 1 - b
        hx, hi = hin[b]
        hx.wait()
        hi.wait()
        if k + 1 < _K:
            if hs[nb] is not None:
                hs[nb].wait()
            hin[nb] = start(k + 1, nb)
        hs[b] = pltpu.async_copy(xbuf.at[b], acc.at[s].at[ibuf.at[b]],
                                 semsc.at[b], add=True)
    for b in range(2):
        if hs[b] is not None:
            hs[b].wait()

    @pl.when(r < _REM)
    def _():
        base = (first + _K) * _C
        pltpu.sync_copy(x_hbm.at[pl.ds(base, _C)], xbuf.at[0])
        pltpu.sync_copy(b_hbm.at[pl.ds(base, _C)], ibuf.at[0])
        pltpu.sync_copy(xbuf.at[0], acc.at[s].at[ibuf.at[0]], add=True)

    @pl.when(r == _NW - 1)
    def _():
        base = _NFULL * _C
        pltpu.sync_copy(x_hbm.at[pl.ds(base, _TAIL)], xtail)
        pltpu.sync_copy(b_hbm.at[pl.ds(base, _TAIL)], itail)
        pltpu.sync_copy(xtail, acc.at[s].at[itail], add=True)

    plsc.subcore_barrier()

    stripe = pl.ds(s * _ROWS, _ROWS)
    iota = lax.iota(jnp.int32, 16)
    for g in range(_ROWS // 16):
        sidx[pl.ds(g * 16, 16)] = s * _ROWS + g * 16 + iota
    hrp = [None, None]
    hrp[0] = pltpu.async_copy(acc.at[0, stripe], rbuf.at[0], semr.at[0])
    hrp[1] = pltpu.async_copy(acc.at[1, stripe], rbuf.at[1], semr.at[1])
    for t in range(2, _NS):
        b = t & 1
        hrp[b].wait()
        hrp[b] = pltpu.async_copy(acc.at[t].at[sidx], rbuf.at[b],
                                  semr.at[b], add=True)
    hrp[0].wait()
    hrp[1].wait()

    def _row(i, _):
        for f in range(_D // 16):
            zbuf[i, pl.ds(f * 16, 16)] = (
                rbuf[0, i, pl.ds(f * 16, 16)] + rbuf[1, i, pl.ds(f * 16, 16)])
        return _

    lax.fori_loop(0, _ROWS, _row, 0)
    pltpu.sync_copy(zbuf, out_hbm.at[pl.ds(c * _S + s * _ROWS, _ROWS)])


_mesh = plsc.VectorSubcoreMesh(
    core_axis_name="c", subcore_axis_name="s",
    num_cores=_NC, num_subcores=_NS)

_seg_sum = pl.kernel(
    _seg_body,
    out_type=jax.ShapeDtypeStruct((_NC * _S, _D), jnp.float32),
    mesh=_mesh,
    scratch_types=[
        pltpu.VMEM((2, _C, _D), jnp.float32),
        pltpu.VMEM((2, _C), jnp.int32),
        pltpu.VMEM((_TAIL, _D), jnp.float32),
        pltpu.VMEM((_TAIL,), jnp.int32),
        pltpu.VMEM((_ROWS, _D), jnp.float32),
        pltpu.VMEM((2, _ROWS, _D), jnp.float32),
        pltpu.VMEM((_ROWS,), jnp.int32),
        pltpu.VMEM_SHARED((_NS, _S, _D), jnp.float32),
        pltpu.SemaphoreType.DMA((2,)),
        pltpu.SemaphoreType.DMA((2,)),
        pltpu.SemaphoreType.DMA((2,)),
        pltpu.SemaphoreType.DMA((2,)),
    ],
)


def _combine_body(p_ref, b_ref, o_ref):
    o_ref[...] = p_ref[:_S, :] + p_ref[_S:, :] + b_ref[0]


def kernel(x, batch, bias):
    b32 = batch.astype(jnp.int32)
    partials = _seg_sum(x, b32)
    bias_v = jnp.asarray(bias, jnp.float32).reshape(1)
    return pl.pallas_call(
        _combine_body,
        out_shape=jax.ShapeDtypeStruct((_S, _D), jnp.float32),
        in_specs=[
            pl.BlockSpec(memory_space=pltpu.VMEM),
            pl.BlockSpec(memory_space=pltpu.SMEM),
        ],
        out_specs=pl.BlockSpec(memory_space=pltpu.VMEM),
    )(partials, bias_v)
